# Initial kernel scaffold; baseline (speedup 1.0000x reference)
import jax
import jax.numpy as jnp
from jax.experimental import pallas as pl


def kernel(x, edge_index, batch, Wl0, bl0, Wr0, lns0, lnb0, Wl1, bl1, Wr1, lns1, lnb1, Wl2, bl2, Wr2, lns2, lnb2, Wc1, bc1, Wc2, bc2, Wc3, bc3):
    raise NotImplementedError("write your pallas kernel here")


# speedup vs baseline: 2.9038x; 2.9038x over previous
import functools

import jax
import jax.numpy as jnp
from jax import lax
from jax.experimental import pallas as pl
from jax.experimental.pallas import tpu as pltpu
from jax.experimental.pallas import tpu_sc as plsc

N = 10000
E = 160000
G = 128
D = 256
F = 128
NS = 16
K = 128
NCHUNK = E // K
CPT = -(-NCHUNK // NS)
RB = 624
RTAIL = N - NS * RB
BN = 1000
NB = N // BN
EPS = 1e-5

_ZCHUNKS = [(t, min(K, RB - t)) for t in range(0, RB, K)]


def _sc_agg_body(h2, srcl, dstl, out, srcv, dstv, rows, acc, sem):
    c = lax.axis_index("c")
    s = lax.axis_index("s")
    r0 = s * RB
    last = s == NS - 1

    def zrow(i, carry):
        for j in range(F // 16):
            rows[i, pl.ds(j * 16, 16)] = jnp.zeros((16,), jnp.float32)
        return carry

    lax.fori_loop(0, K, zrow, 0)

    for t, cnt in _ZCHUNKS:
        pltpu.sync_copy(rows.at[pl.ds(0, cnt)], acc.at[pl.ds(r0 + t, cnt)])

    @pl.when(last)
    def _():
        pltpu.sync_copy(rows.at[pl.ds(0, RTAIL)],
                        acc.at[pl.ds(NS * RB, RTAIL)])

    plsc.subcore_barrier()

    coff = c * N

    def body(k, carry):
        chunk = s + k * NS

        @pl.when(chunk < NCHUNK)
        def _():
            base = chunk * K
            pltpu.sync_copy(srcl.at[pl.ds(base, K)], srcv)
            pltpu.sync_copy(dstl.at[pl.ds(base, K)], dstv)
            for j in range(K // 16):
                sl = pl.ds(j * 16, 16)
                srcv[sl] = srcv[sl] + coff
            pltpu.async_copy(h2.at[srcv], rows, sem).wait()
            pltpu.sync_copy(rows, acc.at[dstv], add=True)

        return carry

    lax.fori_loop(0, CPT, body, 0)
    plsc.subcore_barrier()

    for t, cnt in _ZCHUNKS:
        pltpu.sync_copy(acc.at[pl.ds(r0 + t, cnt)], rows.at[pl.ds(0, cnt)])
        pltpu.sync_copy(rows.at[pl.ds(0, cnt)],
                        out.at[pl.ds(coff + r0 + t, cnt)])

    @pl.when(last)
    def _():
        pltpu.sync_copy(acc.at[pl.ds(NS * RB, RTAIL)], rows.at[pl.ds(0, RTAIL)])
        pltpu.sync_copy(rows.at[pl.ds(0, RTAIL)],
                        out.at[pl.ds(coff + NS * RB, RTAIL)])


def _sc_deg_body(dstl, deg, dstv, zb16, onesv, dacc, sem):
    c = lax.axis_index("c")
    s = lax.axis_index("s")
    r0 = s * RB
    last = s == NS - 1

    def zrow(i, carry):
        for j in range(F // 16):
            sl = pl.ds(j * 16, 16)
            zb16[i, sl] = jnp.zeros((16,), jnp.float32)
            onesv[i, sl] = jnp.full((16,), 1.0, jnp.float32)
        return carry

    lax.fori_loop(0, K, zrow, 0)

    @pl.when(c == 0)
    def _():
        for t, cnt in _ZCHUNKS:
            pltpu.sync_copy(zb16.at[pl.ds(0, cnt)],
                            dacc.at[pl.ds(r0 + t, cnt)])

        @pl.when(last)
        def _():
            pltpu.sync_copy(zb16.at[pl.ds(0, RTAIL)],
                            dacc.at[pl.ds(NS * RB, RTAIL)])

    plsc.subcore_barrier()

    def body(k, carry):
        chunk = s + k * NS

        @pl.when((chunk < NCHUNK) & (c == 0))
        def _():
            base = chunk * K
            pltpu.sync_copy(dstl.at[pl.ds(base, K)], dstv)
            pltpu.sync_copy(onesv, dacc.at[dstv], add=True)

        return carry

    lax.fori_loop(0, CPT, body, 0)
    plsc.subcore_barrier()

    @pl.when(c == 0)
    def _():
        for t, cnt in _ZCHUNKS:
            pltpu.sync_copy(dacc.at[pl.ds(r0 + t, cnt)],
                            zb16.at[pl.ds(0, cnt)])
            pltpu.sync_copy(zb16.at[pl.ds(0, cnt)],
                            deg.at[pl.ds(r0 + t, cnt)])

        @pl.when(last)
        def _():
            pltpu.sync_copy(dacc.at[pl.ds(NS * RB, RTAIL)],
                            zb16.at[pl.ds(0, RTAIL)])
            pltpu.sync_copy(zb16.at[pl.ds(0, RTAIL)],
                            deg.at[pl.ds(NS * RB, RTAIL)])


@functools.lru_cache(maxsize=None)
def _build_sc_kernels():
    mesh = plsc.VectorSubcoreMesh(core_axis_name="c", subcore_axis_name="s",
                                  num_cores=2, num_subcores=NS)
    agg = functools.partial(
        pl.kernel,
        out_type=jax.ShapeDtypeStruct((2 * N, F), jnp.float32),
        mesh=mesh,
        scratch_types=[
            pltpu.VMEM((K,), jnp.int32),
            pltpu.VMEM((K,), jnp.int32),
            pltpu.VMEM((K, F), jnp.float32),
            pltpu.VMEM_SHARED((N, F), jnp.float32),
            pltpu.SemaphoreType.DMA,
        ],
    )(_sc_agg_body)
    degk = functools.partial(
        pl.kernel,
        out_type=jax.ShapeDtypeStruct((N, F), jnp.float32),
        mesh=mesh,
        scratch_types=[
            pltpu.VMEM((K,), jnp.int32),
            pltpu.VMEM((K, F), jnp.float32),
            pltpu.VMEM((K, F), jnp.float32),
            pltpu.VMEM_SHARED((N, F), jnp.float32),
            pltpu.SemaphoreType.DMA,
        ],
    )(_sc_deg_body)
    return agg, degk


def _sc_agg_call(h2, srcl, dstl):
    agg, _ = _build_sc_kernels()
    return agg(h2, srcl, dstl)


def _sc_deg_call(dstl):
    _, degk = _build_sc_kernels()
    return degk(dstl)


def _tc_layer_body(a0, a1, dg, h0, h1, wl, bl, wr, lns, lnb, oa, ob):
    agg = jnp.concatenate([a0[...], a1[...]], axis=1)
    inv = 1.0 / jnp.maximum(dg[...][:, :1], 1.0)
    agg = agg * inv
    hp = jnp.concatenate([h0[...], h1[...]], axis=1)
    h = (jnp.dot(agg, wl[...], preferred_element_type=jnp.float32)
         + bl[...]
         + jnp.dot(hp, wr[...], preferred_element_type=jnp.float32))
    mu = jnp.mean(h, axis=1, keepdims=True)
    var = jnp.mean((h - mu) ** 2, axis=1, keepdims=True)
    h = (h - mu) * lax.rsqrt(var + EPS) * lns[...] + lnb[...]
    h = jnp.maximum(h, 0.0)
    oa[...] = h[:, :F]
    ob[...] = h[:, F:]


_tc_layer = pl.pallas_call(
    _tc_layer_body,
    grid=(NB,),
    in_specs=[
        pl.BlockSpec((BN, F), lambda i: (i, 0)),
        pl.BlockSpec((BN, F), lambda i: (i + NB, 0)),
        pl.BlockSpec((BN, F), lambda i: (i, 0)),
        pl.BlockSpec((BN, F), lambda i: (i, 0)),
        pl.BlockSpec((BN, F), lambda i: (i + NB, 0)),
        pl.BlockSpec((D, D), lambda i: (0, 0)),
        pl.BlockSpec((1, D), lambda i: (0, 0)),
        pl.BlockSpec((D, D), lambda i: (0, 0)),
        pl.BlockSpec((1, D), lambda i: (0, 0)),
        pl.BlockSpec((1, D), lambda i: (0, 0)),
    ],
    out_specs=[
        pl.BlockSpec((BN, F), lambda i: (i, 0)),
        pl.BlockSpec((BN, F), lambda i: (i, 0)),
    ],
    out_shape=[
        jax.ShapeDtypeStruct((N, F), jnp.float32),
        jax.ShapeDtypeStruct((N, F), jnp.float32),
    ],
)


def _pool_mlp_body(ha, hb, bt, wc1, bc1, wc2, bc2, wc3, bc3, out, acc):
    i = pl.program_id(0)

    @pl.when(i == 0)
    def _():
        acc[...] = jnp.full((G, D), -jnp.inf, jnp.float32)

    h = jnp.concatenate([ha[...], hb[...]], axis=1)
    b = bt[...]

    def body(g, carry):
        m = b == g

        @pl.when(jnp.any(m))
        def _():
            vals = jnp.where(m, h, -jnp.inf)
            mx = jnp.max(vals, axis=0)
            cur = acc[pl.ds(g, 1), :]
            acc[pl.ds(g, 1), :] = jnp.maximum(cur, mx[None, :])

        return carry

    lax.fori_loop(0, G, body, 0)

    @pl.when(i == NB - 1)
    def _():
        gv = acc[...]
        gv = jnp.where(gv == -jnp.inf, 0.0, gv)
        z = jnp.maximum(
            jnp.dot(gv, wc1[...], preferred_element_type=jnp.float32)
            + bc1[...], 0.0)
        z = jnp.maximum(
            jnp.dot(z, wc2[...], preferred_element_type=jnp.float32)
            + bc2[...], 0.0)
        out[...] = (jnp.dot(z, wc3[...], preferred_element_type=jnp.float32)
                    + bc3[...])


_pool_mlp = pl.pallas_call(
    _pool_mlp_body,
    grid=(NB,),
    in_specs=[
        pl.BlockSpec((BN, F), lambda i: (i, 0)),
        pl.BlockSpec((BN, F), lambda i: (i + NB, 0)),
        pl.BlockSpec((BN, 1), lambda i: (i, 0)),
        pl.BlockSpec((D, D), lambda i: (0, 0)),
        pl.BlockSpec((1, D), lambda i: (0, 0)),
        pl.BlockSpec((D, G), lambda i: (0, 0)),
        pl.BlockSpec((1, G), lambda i: (0, 0)),
        pl.BlockSpec((G, G), lambda i: (0, 0)),
        pl.BlockSpec((1, G), lambda i: (0, 0)),
    ],
    out_specs=pl.BlockSpec((G, G), lambda i: (0, 0)),
    out_shape=jax.ShapeDtypeStruct((G, G), jnp.float32),
    scratch_shapes=[pltpu.VMEM((G, D), jnp.float32)],
)


def kernel(x, edge_index, batch, Wl0, bl0, Wr0, lns0, lnb0, Wl1, bl1, Wr1,
           lns1, lnb1, Wl2, bl2, Wr2, lns2, lnb2, Wc1, bc1, Wc2, bc2, Wc3,
           bc3):
    src = edge_index[0]
    dst = edge_index[1]

    h2 = jnp.concatenate([x[:, :F], x[:, F:]], axis=0)
    degm = _sc_deg_call(dst)
    layers = [(Wl0, bl0, Wr0, lns0, lnb0), (Wl1, bl1, Wr1, lns1, lnb1),
              (Wl2, bl2, Wr2, lns2, lnb2)]
    for (Wl, bl, Wr, lns, lnb) in layers:
        agg2 = _sc_agg_call(h2, src, dst)
        ha, hb = _tc_layer(agg2, agg2, degm, h2, h2,
                           Wl, bl.reshape(1, D), Wr,
                           lns.reshape(1, D), lnb.reshape(1, D))
        h2 = jnp.concatenate([ha, hb], axis=0)

    batch3 = batch.reshape(N, 1)
    wc3p = jnp.pad(Wc3, ((0, 0), (0, G - 14)))
    bc3p = jnp.pad(bc3, (0, G - 14)).reshape(1, G)
    logits_p = _pool_mlp(h2, h2, batch3,
                         Wc1, bc1.reshape(1, D), Wc2, bc2.reshape(1, G),
                         wc3p, bc3p)
    logits = logits_p[:, :14]
    return (logits[:, :1], logits[:, 1:])


# speedup vs baseline: 3.9719x; 1.3678x over previous
import functools

import jax
import jax.numpy as jnp
from jax import lax
from jax.experimental import pallas as pl
from jax.experimental.pallas import tpu as pltpu
from jax.experimental.pallas import tpu_sc as plsc

N = 10000
E = 160000
G = 128
D = 256
F = 128
NS = 16
K = 80
EPT = E // NS
SEG = 5
SCPT = EPT // (SEG * K)
KD = 128
NCHUNK = E // KD
CPT = -(-NCHUNK // NS)
RB = 624
RTAIL = N - NS * RB
BN = 1000
NB = N // BN
EPS = 1e-5

def _zchunks(step):
    return [(t, min(step, RB - t)) for t in range(0, RB, step)]


def _sc_agg_body(h2, src3, dst3, out, srcall, dstall, rows0, rows1, acc,
                 sem0, sem1):
    c = lax.axis_index("c")
    s = lax.axis_index("s")
    r0 = s * RB
    last = s == NS - 1

    def zrow(i, carry):
        for j in range(F // 16):
            rows0[i, pl.ds(j * 16, 16)] = jnp.zeros((16,), jnp.float32)
        return carry

    lax.fori_loop(0, K, zrow, 0)

    for t, cnt in _zchunks(K):
        pltpu.sync_copy(rows0.at[pl.ds(0, cnt)], acc.at[pl.ds(r0 + t, cnt)])

    @pl.when(last)
    def _():
        pltpu.sync_copy(rows0.at[pl.ds(0, RTAIL)],
                        acc.at[pl.ds(NS * RB, RTAIL)])

    plsc.subcore_barrier()

    bufs = ((rows0, sem0), (rows1, sem1))

    def fire(j, b):
        rows, sem = bufs[b]
        pltpu.async_copy(h2.at[srcall.at[j]], rows, sem)

    def drain(j, b):
        rows, sem = bufs[b]
        pltpu.make_async_copy(h2.at[srcall.at[j]], rows, sem).wait()
        pltpu.sync_copy(rows, acc.at[dstall.at[j]], add=True)

    for g in range(SEG):
        pltpu.sync_copy(src3.at[(c * NS + s) * SEG + g], srcall)
        pltpu.sync_copy(dst3.at[s * SEG + g], dstall)
        fire(0, 0)
        fire(1, 1)

        def body(k2, carry):
            for b in range(2):
                j = k2 * 2 + b

                @pl.when(j < SCPT)
                def _():
                    drain(j, b)

                @pl.when(j + 2 < SCPT)
                def _():
                    fire(j + 2, b)

            return carry

        lax.fori_loop(0, (SCPT + 2) // 2, body, 0)

    plsc.subcore_barrier()

    for t, cnt in _zchunks(K):
        pltpu.sync_copy(acc.at[pl.ds(r0 + t, cnt)], rows0.at[pl.ds(0, cnt)])
        pltpu.sync_copy(rows0.at[pl.ds(0, cnt)],
                        out.at[pl.ds(coff_out(c) + r0 + t, cnt)])

    @pl.when(last)
    def _():
        pltpu.sync_copy(acc.at[pl.ds(NS * RB, RTAIL)],
                        rows0.at[pl.ds(0, RTAIL)])
        pltpu.sync_copy(rows0.at[pl.ds(0, RTAIL)],
                        out.at[pl.ds(coff_out(c) + NS * RB, RTAIL)])


def coff_out(c):
    return c * N


def _sc_deg_body(dstl, deg, dstv, zb16, onesv, dacc, sem):
    c = lax.axis_index("c")
    s = lax.axis_index("s")
    r0 = s * RB
    last = s == NS - 1

    def zrow(i, carry):
        for j in range(F // 16):
            sl = pl.ds(j * 16, 16)
            zb16[i, sl] = jnp.zeros((16,), jnp.float32)
            onesv[i, sl] = jnp.full((16,), 1.0, jnp.float32)
        return carry

    lax.fori_loop(0, KD, zrow, 0)

    @pl.when(c == 0)
    def _():
        for t, cnt in _zchunks(KD):
            pltpu.sync_copy(zb16.at[pl.ds(0, cnt)],
                            dacc.at[pl.ds(r0 + t, cnt)])

        @pl.when(last)
        def _():
            pltpu.sync_copy(zb16.at[pl.ds(0, RTAIL)],
                            dacc.at[pl.ds(NS * RB, RTAIL)])

    plsc.subcore_barrier()

    def body(k, carry):
        chunk = s + k * NS

        @pl.when((chunk < NCHUNK) & (c == 0))
        def _():
            base = chunk * KD
            pltpu.sync_copy(dstl.at[pl.ds(base, KD)], dstv)
            pltpu.sync_copy(onesv, dacc.at[dstv], add=True)

        return carry

    lax.fori_loop(0, CPT, body, 0)
    plsc.subcore_barrier()

    @pl.when(c == 0)
    def _():
        for t, cnt in _zchunks(KD):
            pltpu.sync_copy(dacc.at[pl.ds(r0 + t, cnt)],
                            zb16.at[pl.ds(0, cnt)])
            pltpu.sync_copy(zb16.at[pl.ds(0, cnt)],
                            deg.at[pl.ds(r0 + t, cnt)])

        @pl.when(last)
        def _():
            pltpu.sync_copy(dacc.at[pl.ds(NS * RB, RTAIL)],
                            zb16.at[pl.ds(0, RTAIL)])
            pltpu.sync_copy(zb16.at[pl.ds(0, RTAIL)],
                            deg.at[pl.ds(NS * RB, RTAIL)])


@functools.lru_cache(maxsize=None)
def _build_sc_kernels():
    mesh = plsc.VectorSubcoreMesh(core_axis_name="c", subcore_axis_name="s",
                                  num_cores=2, num_subcores=NS)
    agg = functools.partial(
        pl.kernel,
        out_type=jax.ShapeDtypeStruct((2 * N, F), jnp.float32),
        mesh=mesh,
        scratch_types=[
            pltpu.VMEM((SCPT, K), jnp.int32),
            pltpu.VMEM((SCPT, K), jnp.int32),
            pltpu.VMEM((K, F), jnp.float32),
            pltpu.VMEM((K, F), jnp.float32),
            pltpu.VMEM_SHARED((N, F), jnp.float32),
            pltpu.SemaphoreType.DMA,
            pltpu.SemaphoreType.DMA,
        ],
    )(_sc_agg_body)
    degk = functools.partial(
        pl.kernel,
        out_type=jax.ShapeDtypeStruct((N, F), jnp.float32),
        mesh=mesh,
        scratch_types=[
            pltpu.VMEM((KD,), jnp.int32),
            pltpu.VMEM((KD, F), jnp.float32),
            pltpu.VMEM((KD, F), jnp.float32),
            pltpu.VMEM_SHARED((N, F), jnp.float32),
            pltpu.SemaphoreType.DMA,
        ],
    )(_sc_deg_body)
    return agg, degk


def _sc_agg_call(h2, src3, dst3):
    agg, _ = _build_sc_kernels()
    return agg(h2, src3, dst3)


def _sc_deg_call(dstl):
    _, degk = _build_sc_kernels()
    return degk(dstl)


def _tc_layer_body(a0, a1, dg, h0, h1, wl, bl, wr, lns, lnb, o):
    agg = jnp.concatenate([a0[...], a1[...]], axis=1)
    inv = 1.0 / jnp.maximum(dg[...][:, :1], 1.0)
    agg = agg * inv
    hp = jnp.concatenate([h0[...], h1[...]], axis=1)
    h = (jnp.dot(agg, wl[...], preferred_element_type=jnp.float32)
         + bl[...]
         + jnp.dot(hp, wr[...], preferred_element_type=jnp.float32))
    mu = jnp.mean(h, axis=1, keepdims=True)
    var = jnp.mean((h - mu) ** 2, axis=1, keepdims=True)
    h = (h - mu) * lax.rsqrt(var + EPS) * lns[...] + lnb[...]
    h = jnp.maximum(h, 0.0)
    j = pl.program_id(1)

    @pl.when(j == 0)
    def _():
        o[...] = h[:, :F]

    @pl.when(j == 1)
    def _():
        o[...] = h[:, F:]


_tc_layer = pl.pallas_call(
    _tc_layer_body,
    grid=(NB, 2),
    in_specs=[
        pl.BlockSpec((BN, F), lambda i, j: (i, 0)),
        pl.BlockSpec((BN, F), lambda i, j: (i + NB, 0)),
        pl.BlockSpec((BN, F), lambda i, j: (i, 0)),
        pl.BlockSpec((BN, F), lambda i, j: (i, 0)),
        pl.BlockSpec((BN, F), lambda i, j: (i + NB, 0)),
        pl.BlockSpec((D, D), lambda i, j: (0, 0)),
        pl.BlockSpec((1, D), lambda i, j: (0, 0)),
        pl.BlockSpec((D, D), lambda i, j: (0, 0)),
        pl.BlockSpec((1, D), lambda i, j: (0, 0)),
        pl.BlockSpec((1, D), lambda i, j: (0, 0)),
    ],
    out_specs=pl.BlockSpec((BN, F), lambda i, j: (i + j * NB, 0)),
    out_shape=jax.ShapeDtypeStruct((2 * N, F), jnp.float32),
)


def _pool_mlp_body(ha, hb, bt, wc1, bc1, wc2, bc2, wc3, bc3, out, acc):
    i = pl.program_id(0)

    @pl.when(i == 0)
    def _():
        acc[...] = jnp.full((G, D), -jnp.inf, jnp.float32)

    h = jnp.concatenate([ha[...], hb[...]], axis=1)
    b = bt[...]

    def body(g, carry):
        m = b == g

        @pl.when(jnp.any(m))
        def _():
            vals = jnp.where(m, h, -jnp.inf)
            mx = jnp.max(vals, axis=0)
            cur = acc[pl.ds(g, 1), :]
            acc[pl.ds(g, 1), :] = jnp.maximum(cur, mx[None, :])

        return carry

    lax.fori_loop(0, G, body, 0)

    @pl.when(i == NB - 1)
    def _():
        gv = acc[...]
        gv = jnp.where(gv == -jnp.inf, 0.0, gv)
        z = jnp.maximum(
            jnp.dot(gv, wc1[...], preferred_element_type=jnp.float32)
            + bc1[...], 0.0)
        z = jnp.maximum(
            jnp.dot(z, wc2[...], preferred_element_type=jnp.float32)
            + bc2[...], 0.0)
        out[...] = (jnp.dot(z, wc3[...], preferred_element_type=jnp.float32)
                    + bc3[...])


_pool_mlp = pl.pallas_call(
    _pool_mlp_body,
    grid=(NB,),
    in_specs=[
        pl.BlockSpec((BN, F), lambda i: (i, 0)),
        pl.BlockSpec((BN, F), lambda i: (i + NB, 0)),
        pl.BlockSpec((BN, 1), lambda i: (i, 0)),
        pl.BlockSpec((D, D), lambda i: (0, 0)),
        pl.BlockSpec((1, D), lambda i: (0, 0)),
        pl.BlockSpec((D, G), lambda i: (0, 0)),
        pl.BlockSpec((1, G), lambda i: (0, 0)),
        pl.BlockSpec((G, G), lambda i: (0, 0)),
        pl.BlockSpec((1, G), lambda i: (0, 0)),
    ],
    out_specs=pl.BlockSpec((G, G), lambda i: (0, 0)),
    out_shape=jax.ShapeDtypeStruct((G, G), jnp.float32),
    scratch_shapes=[pltpu.VMEM((G, D), jnp.float32)],
)


def kernel(x, edge_index, batch, Wl0, bl0, Wr0, lns0, lnb0, Wl1, bl1, Wr1,
           lns1, lnb1, Wl2, bl2, Wr2, lns2, lnb2, Wc1, bc1, Wc2, bc2, Wc3,
           bc3):
    src = edge_index[0]
    dst = edge_index[1]
    src3 = jnp.concatenate([src, src + N]).reshape(2 * NS * SEG, SCPT, K)
    dst3 = dst.reshape(NS * SEG, SCPT, K)

    h2 = jnp.concatenate([x[:, :F], x[:, F:]], axis=0)
    degm = _sc_deg_call(dst)
    layers = [(Wl0, bl0, Wr0, lns0, lnb0), (Wl1, bl1, Wr1, lns1, lnb1),
              (Wl2, bl2, Wr2, lns2, lnb2)]
    for (Wl, bl, Wr, lns, lnb) in layers:
        agg2 = _sc_agg_call(h2, src3, dst3)
        h2 = _tc_layer(agg2, agg2, degm, h2, h2,
                       Wl, bl.reshape(1, D), Wr,
                       lns.reshape(1, D), lnb.reshape(1, D))

    batch3 = batch.reshape(N, 1)
    wc3p = jnp.pad(Wc3, ((0, 0), (0, G - 14)))
    bc3p = jnp.pad(bc3, (0, G - 14)).reshape(1, G)
    logits_p = _pool_mlp(h2, h2, batch3,
                         Wc1, bc1.reshape(1, D), Wc2, bc2.reshape(1, G),
                         wc3p, bc3p)
    logits = logits_p[:, :14]
    return (logits[:, :1], logits[:, 1:])
